# baseline (speedup 1.0000x reference)
import functools

import jax
import jax.numpy as jnp
from jax import lax
from jax.experimental import pallas as pl
from jax.experimental.pallas import tpu as pltpu
from jax.experimental.pallas import tpu_sc as plsc

_INFO = plsc.get_sparse_core_info()
_NC = _INFO.num_cores
_NS = _INFO.num_subcores
_NW = _NC * _NS

_N = 16384 * 100
_D = 45
_DP = 48
_PER_W = _N // _NW
_WIN = 2048
_STEPS = _PER_W // _WIN


def _gather_body(x_hbm, di_hbm, out_hbm, idx_v, rows_v, sem):
    wid = lax.axis_index("s") * _NC + lax.axis_index("c")
    wbase = wid * _PER_W

    def step(i, _):
        base = wbase + i * _WIN
        pltpu.sync_copy(x_hbm.at[pl.ds(base, _WIN)], idx_v)
        pltpu.async_copy(di_hbm.at[idx_v], rows_v, sem).wait()
        pltpu.sync_copy(rows_v, out_hbm.at[pl.ds(base, _WIN)])
        return _

    lax.fori_loop(0, _STEPS, step, 0)


@jax.jit
def kernel(x, di):
    xf = x.reshape(_N)
    dip = jnp.pad(di, ((0, 0), (0, _DP - _D)))
    mesh = plsc.VectorSubcoreMesh(core_axis_name="c", subcore_axis_name="s")
    out = pl.kernel(
        _gather_body,
        mesh=mesh,
        out_type=jax.ShapeDtypeStruct((_N, _DP), jnp.float32),
        scratch_types=[
            pltpu.VMEM((_WIN,), jnp.int32),
            pltpu.VMEM((_WIN, _DP), jnp.float32),
            pltpu.SemaphoreType.DMA,
        ],
        compiler_params=pltpu.CompilerParams(use_tc_tiling_on_sc=False),
    )(xf, dip)
    return out.reshape(x.shape[0], x.shape[1], _DP)[..., :_D]


# speedup vs baseline: 2.7383x; 2.7383x over previous
import functools

import jax
import jax.numpy as jnp
from jax import lax
from jax.experimental import pallas as pl
from jax.experimental.pallas import tpu as pltpu
from jax.experimental.pallas import tpu_sc as plsc

_INFO = plsc.get_sparse_core_info()
_NC = _INFO.num_cores
_NS = _INFO.num_subcores
_NW = _NC * _NS
_L = _INFO.num_lanes

_N = 16384 * 100
_D = 45
_DP = 48
_PER_W = _N // _NW

_WIN_A = 2048
_STEPS_A = _PER_W // _WIN_A

_WIN_B = 512
_STEPS_B = _PER_W // _WIN_B
_WB = _WIN_B * _DP


def _gather_body(x_hbm, di_hbm, out_hbm, idx_v, rows_v, sem):
    wid = lax.axis_index("s") * _NC + lax.axis_index("c")
    wbase = wid * _PER_W

    def step(i, _):
        base = wbase + i * _WIN_A
        pltpu.sync_copy(x_hbm.at[pl.ds(base, _WIN_A)], idx_v)
        pltpu.async_copy(di_hbm.at[idx_v], rows_v, sem).wait()
        pltpu.sync_copy(rows_v, out_hbm.at[pl.ds(base, _WIN_A)])
        return _

    lax.fori_loop(0, _STEPS_A, step, 0)


def _plane_body(mid_hbm, out_hbm, vflat, plane_v, idxpat, sflat, sem, sem2):
    sid = lax.axis_index("s")
    wid = sid * _NC + lax.axis_index("c")
    wbase = wid * _PER_W
    soff = sid * _WB
    iota = lax.iota(jnp.int32, _L)

    def build(c, _c):
        k_v = c * _L + iota
        b_v = soff + k_v * _DP
        for j in range(_D):
            idxpat[pl.ds(j * _WIN_B + c * _L, _L)] = b_v + j
        return _c

    lax.fori_loop(0, _WIN_B // _L, build, 0)

    def step(i, _):
        base = wbase + i * _WIN_B
        pltpu.sync_copy(mid_hbm.at[pl.ds(base * _DP, _WB)], vflat)
        pltpu.sync_copy(vflat, sflat.at[pl.ds(soff, _WB)])
        gathers = [
            pltpu.async_copy(
                sflat.at[idxpat.at[pl.ds(j * _WIN_B, _WIN_B)]],
                plane_v.at[pl.ds(j * _WIN_B, _WIN_B)],
                sem,
            )
            for j in range(_D)
        ]
        for g in gathers:
            g.wait()
        writes = [
            pltpu.async_copy(
                plane_v.at[pl.ds(j * _WIN_B, _WIN_B)],
                out_hbm.at[j, pl.ds(base, _WIN_B)],
                sem2,
            )
            for j in range(_D)
        ]
        for w in writes:
            w.wait()
        return _

    lax.fori_loop(0, _STEPS_B, step, 0)


@jax.jit
def kernel(x, di):
    xf = x.reshape(_N)
    dip = jnp.pad(di, ((0, 0), (0, _DP - _D)))
    mesh = plsc.VectorSubcoreMesh(core_axis_name="c", subcore_axis_name="s")
    rows = pl.kernel(
        _gather_body,
        mesh=mesh,
        out_type=jax.ShapeDtypeStruct((_N, _DP), jnp.float32),
        scratch_types=[
            pltpu.VMEM((_WIN_A,), jnp.int32),
            pltpu.VMEM((_WIN_A, _DP), jnp.float32),
            pltpu.SemaphoreType.DMA,
        ],
        compiler_params=pltpu.CompilerParams(use_tc_tiling_on_sc=False),
    )(xf, dip)
    mid = rows.reshape(_N * _DP)
    planes = pl.kernel(
        _plane_body,
        mesh=mesh,
        out_type=jax.ShapeDtypeStruct((_D, _N), jnp.float32),
        scratch_types=[
            pltpu.VMEM((_WB,), jnp.float32),
            pltpu.VMEM((_D * _WIN_B,), jnp.float32),
            pltpu.VMEM((_D * _WIN_B,), jnp.int32),
            pltpu.VMEM_SHARED((_NS * _WB,), jnp.float32),
            pltpu.SemaphoreType.DMA,
            pltpu.SemaphoreType.DMA,
        ],
        compiler_params=pltpu.CompilerParams(use_tc_tiling_on_sc=False),
    )(mid)
    return planes.reshape(_D, x.shape[0], x.shape[1]).transpose(1, 2, 0)


# speedup vs baseline: 2.7512x; 1.0047x over previous
import functools

import jax
import jax.numpy as jnp
from jax import lax
from jax.experimental import pallas as pl
from jax.experimental.pallas import tpu as pltpu
from jax.experimental.pallas import tpu_sc as plsc

_INFO = plsc.get_sparse_core_info()
_NC = _INFO.num_cores
_NS = _INFO.num_subcores
_NW = _NC * _NS
_L = _INFO.num_lanes

_N = 16384 * 100
_D = 45
_DP = 48
_PER_W = _N // _NW

_WIN_A = 2048
_STEPS_A = _PER_W // _WIN_A

_WIN_B = 640
_STEPS_B = _PER_W // _WIN_B
_WB = _WIN_B * _DP
_PB = _WIN_B * _D


def _gather_body(x_hbm, di_hbm, out_hbm, idx_v, rows_v, sem):
    wid = lax.axis_index("s") * _NC + lax.axis_index("c")
    wbase = wid * _PER_W

    def step(i, _):
        base = wbase + i * _WIN_A
        pltpu.sync_copy(x_hbm.at[pl.ds(base, _WIN_A)], idx_v)
        pltpu.async_copy(di_hbm.at[idx_v], rows_v, sem).wait()
        pltpu.sync_copy(rows_v, out_hbm.at[pl.ds(base, _WIN_A)])
        return _

    lax.fori_loop(0, _STEPS_A, step, 0)


def _plane_body(mid_hbm, out_hbm, vflat, plane_v, idxpat, sflat, sem, sem2):
    sid = lax.axis_index("s")
    wid = sid * _NC + lax.axis_index("c")
    wbase = wid * _PER_W
    soff = sid * _WB
    iota = lax.iota(jnp.int32, _L)

    def build(c, _c):
        k_v = c * _L + iota
        b_v = soff + k_v * _DP
        for j in range(_D):
            idxpat[pl.ds(j * _WIN_B + c * _L, _L)] = b_v + j
        return _c

    lax.fori_loop(0, _WIN_B // _L, build, 0)

    def step(i, _):
        base = wbase + i * _WIN_B
        pltpu.sync_copy(mid_hbm.at[pl.ds(base * _DP, _WB)], vflat)
        pltpu.sync_copy(vflat, sflat.at[pl.ds(soff, _WB)])
        pltpu.async_copy(sflat.at[idxpat], plane_v, sem).wait()
        writes = [
            pltpu.async_copy(
                plane_v.at[pl.ds(j * _WIN_B, _WIN_B)],
                out_hbm.at[j, pl.ds(base, _WIN_B)],
                sem2,
            )
            for j in range(_D)
        ]
        for w in writes:
            w.wait()
        return _

    lax.fori_loop(0, _STEPS_B, step, 0)


@jax.jit
def kernel(x, di):
    xf = x.reshape(_N)
    dip = jnp.pad(di, ((0, 0), (0, _DP - _D)))
    mesh = plsc.VectorSubcoreMesh(core_axis_name="c", subcore_axis_name="s")
    rows = pl.kernel(
        _gather_body,
        mesh=mesh,
        out_type=jax.ShapeDtypeStruct((_N, _DP), jnp.float32),
        scratch_types=[
            pltpu.VMEM((_WIN_A,), jnp.int32),
            pltpu.VMEM((_WIN_A, _DP), jnp.float32),
            pltpu.SemaphoreType.DMA,
        ],
        compiler_params=pltpu.CompilerParams(use_tc_tiling_on_sc=False),
    )(xf, dip)
    mid = rows.reshape(_N * _DP)
    planes = pl.kernel(
        _plane_body,
        mesh=mesh,
        out_type=jax.ShapeDtypeStruct((_D, _N), jnp.float32),
        scratch_types=[
            pltpu.VMEM((_WB,), jnp.float32),
            pltpu.VMEM((_PB,), jnp.float32),
            pltpu.VMEM((_PB,), jnp.int32),
            pltpu.VMEM_SHARED((_NS * _WB,), jnp.float32),
            pltpu.SemaphoreType.DMA,
            pltpu.SemaphoreType.DMA,
        ],
        compiler_params=pltpu.CompilerParams(use_tc_tiling_on_sc=False),
    )(mid)
    return planes.reshape(_D, x.shape[0], x.shape[1]).transpose(1, 2, 0)


# speedup vs baseline: 2.7674x; 1.0059x over previous
import functools

import jax
import jax.numpy as jnp
from jax import lax
from jax.experimental import pallas as pl
from jax.experimental.pallas import tpu as pltpu
from jax.experimental.pallas import tpu_sc as plsc

_INFO = plsc.get_sparse_core_info()
_NC = _INFO.num_cores
_NS = _INFO.num_subcores
_NW = _NC * _NS
_L = _INFO.num_lanes

_N = 16384 * 100
_D = 45
_DP = 48
_PER_W = _N // _NW

_WIN_A = 2048
_STEPS_A = _PER_W // _WIN_A

_WIN_B = 512
_STEPS_B = _PER_W // _WIN_B
_WB = _WIN_B * _DP
_PB = _WIN_B * _D


def _gather_body(x_hbm, di_hbm, out_hbm, idx_v, rows_v, sem):
    wid = lax.axis_index("s") * _NC + lax.axis_index("c")
    wbase = wid * _PER_W

    def step(i, _):
        base = wbase + i * _WIN_A
        pltpu.sync_copy(x_hbm.at[pl.ds(base, _WIN_A)], idx_v)
        pltpu.async_copy(di_hbm.at[idx_v], rows_v, sem).wait()
        pltpu.sync_copy(rows_v, out_hbm.at[pl.ds(base, _WIN_A)])
        return _

    lax.fori_loop(0, _STEPS_A, step, 0)


def _plane_body(mid_hbm, out_hbm, vflat, plane_v, idxpat, sflat, sem, semw0, semw1):
    sid = lax.axis_index("s")
    wid = sid * _NC + lax.axis_index("c")
    wbase = wid * _PER_W
    soff = sid * _WB
    iota = lax.iota(jnp.int32, _L)

    def build(c, _c):
        k_v = c * _L + iota
        b_v = soff + k_v * _DP
        for j in range(_D):
            idxpat[pl.ds(j * _WIN_B + c * _L, _L)] = b_v + j
        return _c

    lax.fori_loop(0, _WIN_B // _L, build, 0)

    def window(i, parity, semw, drain):
        poff = parity * _PB
        base = wbase + i * _WIN_B
        pltpu.sync_copy(mid_hbm.at[pl.ds(base * _DP, _WB)], vflat)
        pltpu.sync_copy(vflat, sflat.at[pl.ds(soff, _WB)])
        if drain:
            for j in range(_D):
                pltpu.make_async_copy(
                    plane_v.at[pl.ds(poff + j * _WIN_B, _WIN_B)],
                    out_hbm.at[j, pl.ds(base, _WIN_B)],
                    semw,
                ).wait()
        pltpu.async_copy(
            sflat.at[idxpat], plane_v.at[pl.ds(poff, _PB)], sem
        ).wait()
        for j in range(_D):
            pltpu.async_copy(
                plane_v.at[pl.ds(poff + j * _WIN_B, _WIN_B)],
                out_hbm.at[j, pl.ds(base, _WIN_B)],
                semw,
            )

    window(0, 0, semw0, drain=False)
    window(1, 1, semw1, drain=False)

    def step(p, _):
        window(2 * p, 0, semw0, drain=True)
        window(2 * p + 1, 1, semw1, drain=True)
        return _

    lax.fori_loop(1, _STEPS_B // 2, step, 0)

    for parity, semw in ((0, semw0), (1, semw1)):
        poff = parity * _PB
        for j in range(_D):
            pltpu.make_async_copy(
                plane_v.at[pl.ds(poff + j * _WIN_B, _WIN_B)],
                out_hbm.at[j, pl.ds(wbase, _WIN_B)],
                semw,
            ).wait()


@jax.jit
def kernel(x, di):
    xf = x.reshape(_N)
    dip = jnp.pad(di, ((0, 0), (0, _DP - _D)))
    mesh = plsc.VectorSubcoreMesh(core_axis_name="c", subcore_axis_name="s")
    rows = pl.kernel(
        _gather_body,
        mesh=mesh,
        out_type=jax.ShapeDtypeStruct((_N, _DP), jnp.float32),
        scratch_types=[
            pltpu.VMEM((_WIN_A,), jnp.int32),
            pltpu.VMEM((_WIN_A, _DP), jnp.float32),
            pltpu.SemaphoreType.DMA,
        ],
        compiler_params=pltpu.CompilerParams(use_tc_tiling_on_sc=False),
    )(xf, dip)
    mid = rows.reshape(_N * _DP)
    planes = pl.kernel(
        _plane_body,
        mesh=mesh,
        out_type=jax.ShapeDtypeStruct((_D, _N), jnp.float32),
        scratch_types=[
            pltpu.VMEM((_WB,), jnp.float32),
            pltpu.VMEM((2 * _PB,), jnp.float32),
            pltpu.VMEM((_PB,), jnp.int32),
            pltpu.VMEM_SHARED((_NS * _WB,), jnp.float32),
            pltpu.SemaphoreType.DMA,
            pltpu.SemaphoreType.DMA,
            pltpu.SemaphoreType.DMA,
        ],
        compiler_params=pltpu.CompilerParams(use_tc_tiling_on_sc=False),
    )(mid)
    return planes.reshape(_D, x.shape[0], x.shape[1]).transpose(1, 2, 0)
